# Initial kernel scaffold; baseline (speedup 1.0000x reference)
import jax
import jax.numpy as jnp
from jax.experimental import pallas as pl


def kernel(node_attr, edge_index, edge_attr, W1, b1, W2, b2, W3, b3):
    raise NotImplementedError("write your pallas kernel here")


# speedup vs baseline: 35.0640x; 35.0640x over previous
import dataclasses
import functools

import jax
import jax.numpy as jnp
from jax import lax
from jax.experimental import pallas as pl
from jax.experimental.pallas import tpu as pltpu
from jax.experimental.pallas import tpu_sc as plsc

N = 50000
E = 1600000
NC = 2
NS = 16
NW = NC * NS
EPW = E // NW
CHA = 2000
CHB = 2000
NPAD = 50176
NP2 = 2 * NPAD
BT = 6400
BTC = 6272


def _sc_compiler_params():
    cp = pltpu.CompilerParams()
    if "needs_layout_passes" in pltpu.CompilerParams.__dataclass_fields__:
        cp = dataclasses.replace(cp, needs_layout_passes=False)
    return cp


def _mass_product(node_flat, src, dst):
    mesh = plsc.VectorSubcoreMesh(core_axis_name="c", subcore_axis_name="s")

    @functools.partial(
        pl.kernel,
        out_type=jax.ShapeDtypeStruct((E,), jnp.float32),
        mesh=mesh,
        scratch_types=[
            pltpu.VMEM((N,), jnp.float32),
            pltpu.VMEM((CHA,), jnp.int32),
            pltpu.VMEM((CHA,), jnp.int32),
            pltpu.VMEM((CHA,), jnp.float32),
        ],
        compiler_params=_sc_compiler_params(),
    )
    def run(node_hbm, src_hbm, dst_hbm, mm_hbm, node_v, src_v, dst_v, mm_v):
        wid = lax.axis_index("s") * NC + lax.axis_index("c")
        base0 = wid * EPW
        pltpu.sync_copy(node_hbm, node_v)

        @pl.loop(0, EPW, step=CHA)
        def _(j):
            base = base0 + j
            pltpu.sync_copy(src_hbm.at[pl.ds(base, CHA)], src_v)
            pltpu.sync_copy(dst_hbm.at[pl.ds(base, CHA)], dst_v)

            @pl.loop(0, CHA, step=16)
            def _(g):
                sv = src_v[pl.ds(g, 16)]
                dv = dst_v[pl.ds(g, 16)]
                ms = plsc.load_gather(node_v, [sv])
                md = plsc.load_gather(node_v, [dv])
                mm_v[pl.ds(g, 16)] = ms * md

            pltpu.sync_copy(mm_v, mm_hbm.at[pl.ds(base, CHA)])

    return run(node_flat, src, dst)


def _mlp_body(x_ref, w1t_ref, b1_ref, w2t_ref, b2_ref, w3t_ref, b3_ref,
              fx_ref, fy_ref):
    x = x_ref[0:1, :]
    y = x_ref[1:2, :]
    r2 = x * x + y * y + 1e-12
    logr = 0.5 * jnp.log(r2)
    theta = jnp.arctan2(y, x)
    s = jnp.concatenate([logr, theta], axis=0)
    h1 = jnp.tanh(
        jnp.dot(w1t_ref[...], s, preferred_element_type=jnp.float32)
        + b1_ref[...])
    h2 = jnp.tanh(
        jnp.dot(w2t_ref[...], h1.astype(jnp.bfloat16),
                preferred_element_type=jnp.float32)
        + b2_ref[...])
    f = (jnp.dot(w3t_ref[...], h2, preferred_element_type=jnp.float32)
         + b3_ref[...])
    r = jnp.exp(f[0:1, :])
    th = f[1:2, :]
    fx_ref[...] = r * jnp.cos(th)
    fy_ref[...] = r * jnp.sin(th)


def _edge_mlp(x_t, w1t, b1c, w2tb, b2c, w3t, b3c):
    grid = (E // BT,)
    full = lambda shape: pl.BlockSpec(shape, lambda i: (0, 0))
    return pl.pallas_call(
        _mlp_body,
        grid=grid,
        in_specs=[
            pl.BlockSpec((2, BT), lambda i: (0, i)),
            full((128, 2)),
            full((128, 1)),
            full((128, 128)),
            full((128, 1)),
            full((2, 128)),
            full((2, 1)),
        ],
        out_specs=[
            pl.BlockSpec((1, BT), lambda i: (0, i)),
            pl.BlockSpec((1, BT), lambda i: (0, i)),
        ],
        out_shape=[
            jax.ShapeDtypeStruct((1, E), jnp.float32),
            jax.ShapeDtypeStruct((1, E), jnp.float32),
        ],
    )(x_t, w1t, b1c, w2tb, b2c, w3t, b3c)


def _scatter_partials(dst, mm, fx, fy, zeros):
    mesh = plsc.VectorSubcoreMesh(core_axis_name="c", subcore_axis_name="s")

    @functools.partial(
        pl.kernel,
        out_type=jax.ShapeDtypeStruct((NW, NP2), jnp.float32),
        mesh=mesh,
        scratch_types=[
            pltpu.VMEM((NP2,), jnp.float32),
            pltpu.VMEM((CHB,), jnp.int32),
            pltpu.VMEM((CHB,), jnp.float32),
            pltpu.VMEM((CHB,), jnp.float32),
            pltpu.VMEM((CHB,), jnp.float32),
        ],
        compiler_params=_sc_compiler_params(),
    )
    def run(dst_hbm, mm_hbm, fx_hbm, fy_hbm, zero_hbm, out_hbm,
            acc_v, dst_v, mm_v, fx_v, fy_v):
        wid = lax.axis_index("s") * NC + lax.axis_index("c")
        base0 = wid * EPW
        pltpu.sync_copy(zero_hbm, acc_v)

        @pl.loop(0, EPW, step=CHB)
        def _(j):
            base = base0 + j
            pltpu.sync_copy(dst_hbm.at[pl.ds(base, CHB)], dst_v)
            pltpu.sync_copy(mm_hbm.at[pl.ds(base, CHB)], mm_v)
            pltpu.sync_copy(fx_hbm.at[pl.ds(base, CHB)], fx_v)
            pltpu.sync_copy(fy_hbm.at[pl.ds(base, CHB)], fy_v)

            @pl.loop(0, CHB, step=16)
            def _(g):
                d = dst_v[pl.ds(g, 16)]
                m = mm_v[pl.ds(g, 16)]
                vx = fx_v[pl.ds(g, 16)] * m
                vy = fy_v[pl.ds(g, 16)] * m
                plsc.addupdate_scatter(acc_v, [d], vx)
                plsc.addupdate_scatter(acc_v, [d + NPAD], vy)

        pltpu.sync_copy(acc_v, out_hbm.at[wid])

    return run(dst, mm, fx, fy, zeros)


def _reduce_body(p_ref, o_ref):
    o_ref[...] = jnp.sum(p_ref[...], axis=0, keepdims=True)


def _reduce_partials(partials):
    grid = (NP2 // BTC,)
    return pl.pallas_call(
        _reduce_body,
        grid=grid,
        in_specs=[pl.BlockSpec((NW, BTC), lambda i: (0, i))],
        out_specs=pl.BlockSpec((1, BTC), lambda i: (0, i)),
        out_shape=jax.ShapeDtypeStruct((1, NP2), jnp.float32),
    )(partials)


def kernel(node_attr, edge_index, edge_attr, W1, b1, W2, b2, W3, b3):
    node_flat = node_attr.reshape(N)
    src = edge_index[0]
    dst = edge_index[1]
    x_t = edge_attr.T
    w1t = W1.T
    b1c = b1.reshape(128, 1)
    w2tb = W2.T.astype(jnp.bfloat16)
    b2c = b2.reshape(128, 1)
    w3t = W3.T
    b3c = b3.reshape(2, 1)
    zeros = jnp.zeros((NP2,), jnp.float32)

    mm = _mass_product(node_flat, src, dst)
    fx, fy = _edge_mlp(x_t, w1t, b1c, w2tb, b2c, w3t, b3c)
    partials = _scatter_partials(dst, mm, fx.reshape(E), fy.reshape(E), zeros)
    red = _reduce_partials(partials)
    out = jnp.stack([red[0, :N], red[0, NPAD:NPAD + N]], axis=1)
    return out


# speedup vs baseline: 38.3144x; 1.0927x over previous
import dataclasses
import functools

import jax
import jax.numpy as jnp
from jax import lax
from jax.experimental import pallas as pl
from jax.experimental.pallas import tpu as pltpu
from jax.experimental.pallas import tpu_sc as plsc

N = 50000
E = 1600000
NC = 2
NS = 16
NW = NC * NS
EPW = E // NW
CHA = 2000
CHB = 2000
NPAD = 50176
NP2 = 2 * NPAD
BT = 6400
BTC = 6272


def _sc_compiler_params():
    cp = pltpu.CompilerParams()
    if "needs_layout_passes" in pltpu.CompilerParams.__dataclass_fields__:
        cp = dataclasses.replace(cp, needs_layout_passes=False)
    return cp


def _mass_product(node_flat, src, dst):
    mesh = plsc.VectorSubcoreMesh(core_axis_name="c", subcore_axis_name="s")

    @functools.partial(
        pl.kernel,
        out_type=jax.ShapeDtypeStruct((E,), jnp.float32),
        mesh=mesh,
        scratch_types=[
            pltpu.VMEM((N,), jnp.float32),
            pltpu.VMEM((CHA,), jnp.int32),
            pltpu.VMEM((CHA,), jnp.int32),
            pltpu.VMEM((CHA,), jnp.float32),
        ],
        compiler_params=_sc_compiler_params(),
    )
    def run(node_hbm, src_hbm, dst_hbm, mm_hbm, node_v, src_v, dst_v, mm_v):
        wid = lax.axis_index("s") * NC + lax.axis_index("c")
        base0 = wid * EPW
        pltpu.sync_copy(node_hbm, node_v)

        @pl.loop(0, EPW, step=CHA)
        def _(j):
            base = base0 + j
            pltpu.sync_copy(src_hbm.at[pl.ds(base, CHA)], src_v)
            pltpu.sync_copy(dst_hbm.at[pl.ds(base, CHA)], dst_v)

            @pl.loop(0, CHA, step=16)
            def _(g):
                sv = src_v[pl.ds(g, 16)]
                dv = dst_v[pl.ds(g, 16)]
                ms = plsc.load_gather(node_v, [sv])
                md = plsc.load_gather(node_v, [dv])
                mm_v[pl.ds(g, 16)] = ms * md

            pltpu.sync_copy(mm_v, mm_hbm.at[pl.ds(base, CHA)])

    return run(node_flat, src, dst)


def _mlp_body(x_ref, w1t_ref, b1_ref, w2t_ref, b2_ref, w3t_ref, b3_ref,
              lr_ref, th_ref):
    x = x_ref[0:1, :]
    y = x_ref[1:2, :]
    r2 = x * x + y * y + 1e-12
    logr = 0.5 * jnp.log(r2)
    theta = jnp.arctan2(y, x)
    s = jnp.concatenate([logr, theta], axis=0)
    h1 = jnp.tanh(
        jnp.dot(w1t_ref[...], s, preferred_element_type=jnp.float32)
        + b1_ref[...])
    h2 = jnp.tanh(
        jnp.dot(w2t_ref[...], h1.astype(jnp.bfloat16),
                preferred_element_type=jnp.float32)
        + b2_ref[...])
    f = (jnp.dot(w3t_ref[...], h2, preferred_element_type=jnp.float32)
         + b3_ref[...])
    lr_ref[...] = f[0:1, :]
    th_ref[...] = f[1:2, :]


def _edge_mlp(x_t, w1t, b1c, w2tb, b2c, w3t, b3c):
    grid = (E // BT,)
    full = lambda shape: pl.BlockSpec(shape, lambda i: (0, 0))
    return pl.pallas_call(
        _mlp_body,
        grid=grid,
        in_specs=[
            pl.BlockSpec((2, BT), lambda i: (0, i)),
            full((128, 2)),
            full((128, 1)),
            full((128, 128)),
            full((128, 1)),
            full((2, 128)),
            full((2, 1)),
        ],
        out_specs=[
            pl.BlockSpec((1, BT), lambda i: (0, i)),
            pl.BlockSpec((1, BT), lambda i: (0, i)),
        ],
        out_shape=[
            jax.ShapeDtypeStruct((1, E), jnp.float32),
            jax.ShapeDtypeStruct((1, E), jnp.float32),
        ],
    )(x_t, w1t, b1c, w2tb, b2c, w3t, b3c)


E8 = E // 8
BP = 2560


def _polar_body(lr_ref, th_ref, fx_ref, fy_ref):
    r = jnp.exp(lr_ref[...])
    th = th_ref[...]
    fx_ref[...] = r * jnp.cos(th)
    fy_ref[...] = r * jnp.sin(th)


def _polar(lr8, th8):
    grid = ((E8 + BP - 1) // BP,)
    return pl.pallas_call(
        _polar_body,
        grid=grid,
        in_specs=[
            pl.BlockSpec((8, BP), lambda i: (0, i)),
            pl.BlockSpec((8, BP), lambda i: (0, i)),
        ],
        out_specs=[
            pl.BlockSpec((8, BP), lambda i: (0, i)),
            pl.BlockSpec((8, BP), lambda i: (0, i)),
        ],
        out_shape=[
            jax.ShapeDtypeStruct((8, E8), jnp.float32),
            jax.ShapeDtypeStruct((8, E8), jnp.float32),
        ],
    )(lr8, th8)


def _scatter_partials(dst, mm, fx, fy, zeros):
    mesh = plsc.VectorSubcoreMesh(core_axis_name="c", subcore_axis_name="s")

    @functools.partial(
        pl.kernel,
        out_type=jax.ShapeDtypeStruct((NW, NP2), jnp.float32),
        mesh=mesh,
        scratch_types=[
            pltpu.VMEM((NP2,), jnp.float32),
            pltpu.VMEM((CHB,), jnp.int32),
            pltpu.VMEM((CHB,), jnp.float32),
            pltpu.VMEM((CHB,), jnp.float32),
            pltpu.VMEM((CHB,), jnp.float32),
        ],
        compiler_params=_sc_compiler_params(),
    )
    def run(dst_hbm, mm_hbm, fx_hbm, fy_hbm, zero_hbm, out_hbm,
            acc_v, dst_v, mm_v, fx_v, fy_v):
        wid = lax.axis_index("s") * NC + lax.axis_index("c")
        base0 = wid * EPW
        pltpu.sync_copy(zero_hbm, acc_v)

        @pl.loop(0, EPW, step=CHB)
        def _(j):
            base = base0 + j
            pltpu.sync_copy(dst_hbm.at[pl.ds(base, CHB)], dst_v)
            pltpu.sync_copy(mm_hbm.at[pl.ds(base, CHB)], mm_v)
            pltpu.sync_copy(fx_hbm.at[pl.ds(base, CHB)], fx_v)
            pltpu.sync_copy(fy_hbm.at[pl.ds(base, CHB)], fy_v)

            @pl.loop(0, CHB, step=16)
            def _(g):
                d = dst_v[pl.ds(g, 16)]
                m = mm_v[pl.ds(g, 16)]
                vx = fx_v[pl.ds(g, 16)] * m
                vy = fy_v[pl.ds(g, 16)] * m
                plsc.addupdate_scatter(acc_v, [d], vx)
                plsc.addupdate_scatter(acc_v, [d + NPAD], vy)

        pltpu.sync_copy(acc_v, out_hbm.at[wid])

    return run(dst, mm, fx, fy, zeros)


def _reduce_body(p_ref, o_ref):
    o_ref[...] = jnp.sum(p_ref[...], axis=0, keepdims=True)


def _reduce_partials(partials):
    grid = (NP2 // BTC,)
    return pl.pallas_call(
        _reduce_body,
        grid=grid,
        in_specs=[pl.BlockSpec((NW, BTC), lambda i: (0, i))],
        out_specs=pl.BlockSpec((1, BTC), lambda i: (0, i)),
        out_shape=jax.ShapeDtypeStruct((1, NP2), jnp.float32),
    )(partials)


def kernel(node_attr, edge_index, edge_attr, W1, b1, W2, b2, W3, b3):
    node_flat = node_attr.reshape(N)
    src = edge_index[0]
    dst = edge_index[1]
    x_t = edge_attr.T
    w1t = W1.T
    b1c = b1.reshape(128, 1)
    w2tb = W2.T.astype(jnp.bfloat16)
    b2c = b2.reshape(128, 1)
    w3t = W3.T
    b3c = b3.reshape(2, 1)
    zeros = jnp.zeros((NP2,), jnp.float32)

    mm = _mass_product(node_flat, src, dst)
    lr, th = _edge_mlp(x_t, w1t, b1c, w2tb, b2c, w3t, b3c)
    fx, fy = _polar(lr.reshape(8, E8), th.reshape(8, E8))
    partials = _scatter_partials(dst, mm, fx.reshape(E), fy.reshape(E), zeros)
    red = _reduce_partials(partials)
    out = jnp.stack([red[0, :N], red[0, NPAD:NPAD + N]], axis=1)
    return out


# speedup vs baseline: 39.1233x; 1.0211x over previous
import dataclasses
import functools

import jax
import jax.numpy as jnp
from jax import lax
from jax.experimental import pallas as pl
from jax.experimental.pallas import tpu as pltpu
from jax.experimental.pallas import tpu_sc as plsc

N = 50000
E = 1600000
NC = 2
NS = 16
NW = NC * NS
EPW = E // NW
CHA = 2000
CHB = 2000
NPAD = 50176
NP2 = 2 * NPAD
BT = 6400
BTC = 6272


def _sc_compiler_params():
    cp = pltpu.CompilerParams()
    if "needs_layout_passes" in pltpu.CompilerParams.__dataclass_fields__:
        cp = dataclasses.replace(cp, needs_layout_passes=False)
    return cp


def _mass_product(node_flat, src, dst):
    mesh = plsc.VectorSubcoreMesh(core_axis_name="c", subcore_axis_name="s")

    @functools.partial(
        pl.kernel,
        out_type=jax.ShapeDtypeStruct((E,), jnp.float32),
        mesh=mesh,
        scratch_types=[
            pltpu.VMEM((N,), jnp.float32),
            pltpu.VMEM((CHA,), jnp.int32),
            pltpu.VMEM((CHA,), jnp.int32),
            pltpu.VMEM((CHA,), jnp.float32),
        ],
        compiler_params=_sc_compiler_params(),
    )
    def run(node_hbm, src_hbm, dst_hbm, mm_hbm, node_v, src_v, dst_v, mm_v):
        wid = lax.axis_index("s") * NC + lax.axis_index("c")
        base0 = wid * EPW
        pltpu.sync_copy(node_hbm, node_v)

        @pl.loop(0, EPW, step=CHA)
        def _(j):
            base = base0 + j
            pltpu.sync_copy(src_hbm.at[pl.ds(base, CHA)], src_v)
            pltpu.sync_copy(dst_hbm.at[pl.ds(base, CHA)], dst_v)

            @pl.loop(0, CHA, step=16)
            def _(g):
                sv = src_v[pl.ds(g, 16)]
                dv = dst_v[pl.ds(g, 16)]
                ms = plsc.load_gather(node_v, [sv])
                md = plsc.load_gather(node_v, [dv])
                mm_v[pl.ds(g, 16)] = ms * md

            pltpu.sync_copy(mm_v, mm_hbm.at[pl.ds(base, CHA)])

    return run(node_flat, src, dst)


GC = 2000
NB = E // (8 * GC)


def _mlp_body(x_ref, y_ref, w1b_ref, b1_ref, w2t_ref, b2_ref, w3t_ref, b3_ref,
              fx_ref, fy_ref):
    x8 = x_ref[0]
    y8 = y_ref[0]
    r2 = x8 * x8 + y8 * y8 + 1e-12
    logr = 0.5 * jnp.log(r2)
    theta = jnp.arctan2(y8, x8)
    s16 = jnp.concatenate([logr, theta], axis=0).astype(jnp.bfloat16)
    h1 = jnp.tanh(
        jnp.dot(w1b_ref[...], s16, preferred_element_type=jnp.float32)
        + b1_ref[...])
    h1b = h1.astype(jnp.bfloat16)
    lrs, ths = [], []
    for r in range(8):
        h2r = jnp.tanh(
            jnp.dot(w2t_ref[...], h1b[128 * r:128 * (r + 1)],
                    preferred_element_type=jnp.float32)
            + b2_ref[...])
        fr = (jnp.dot(w3t_ref[...], h2r, preferred_element_type=jnp.float32)
              + b3_ref[...])
        lrs.append(fr[0:1])
        ths.append(fr[1:2])
    lr8 = jnp.concatenate(lrs, axis=0)
    th8 = jnp.concatenate(ths, axis=0)
    rad = jnp.exp(lr8)
    fx_ref[0] = rad * jnp.cos(th8)
    fy_ref[0] = rad * jnp.sin(th8)


def _edge_mlp(xs3, ys3, w1big, b1big, w2tb, b2c, w3t, b3c):
    grid = (NB,)
    full = lambda shape: pl.BlockSpec(shape, lambda i: tuple(0 for _ in shape))
    return pl.pallas_call(
        _mlp_body,
        grid=grid,
        in_specs=[
            pl.BlockSpec((1, 8, GC), lambda i: (i, 0, 0)),
            pl.BlockSpec((1, 8, GC), lambda i: (i, 0, 0)),
            full((1024, 16)),
            full((1024, 1)),
            full((128, 128)),
            full((128, 1)),
            full((2, 128)),
            full((2, 1)),
        ],
        out_specs=[
            pl.BlockSpec((1, 8, GC), lambda i: (i, 0, 0)),
            pl.BlockSpec((1, 8, GC), lambda i: (i, 0, 0)),
        ],
        out_shape=[
            jax.ShapeDtypeStruct((NB, 8, GC), jnp.float32),
            jax.ShapeDtypeStruct((NB, 8, GC), jnp.float32),
        ],
    )(xs3, ys3, w1big, b1big, w2tb, b2c, w3t, b3c)


def _scatter_partials(dst, mm, fx, fy, zeros):
    mesh = plsc.VectorSubcoreMesh(core_axis_name="c", subcore_axis_name="s")

    @functools.partial(
        pl.kernel,
        out_type=jax.ShapeDtypeStruct((NW, NP2), jnp.float32),
        mesh=mesh,
        scratch_types=[
            pltpu.VMEM((NP2,), jnp.float32),
            pltpu.VMEM((CHB,), jnp.int32),
            pltpu.VMEM((CHB,), jnp.float32),
            pltpu.VMEM((CHB,), jnp.float32),
            pltpu.VMEM((CHB,), jnp.float32),
        ],
        compiler_params=_sc_compiler_params(),
    )
    def run(dst_hbm, mm_hbm, fx_hbm, fy_hbm, zero_hbm, out_hbm,
            acc_v, dst_v, mm_v, fx_v, fy_v):
        wid = lax.axis_index("s") * NC + lax.axis_index("c")
        base0 = wid * EPW
        pltpu.sync_copy(zero_hbm, acc_v)

        @pl.loop(0, EPW, step=CHB)
        def _(j):
            base = base0 + j
            pltpu.sync_copy(dst_hbm.at[pl.ds(base, CHB)], dst_v)
            pltpu.sync_copy(mm_hbm.at[pl.ds(base, CHB)], mm_v)
            pltpu.sync_copy(fx_hbm.at[pl.ds(base, CHB)], fx_v)
            pltpu.sync_copy(fy_hbm.at[pl.ds(base, CHB)], fy_v)

            @pl.loop(0, CHB, step=16)
            def _(g):
                d = dst_v[pl.ds(g, 16)]
                m = mm_v[pl.ds(g, 16)]
                vx = fx_v[pl.ds(g, 16)] * m
                vy = fy_v[pl.ds(g, 16)] * m
                plsc.addupdate_scatter(acc_v, [d], vx)
                plsc.addupdate_scatter(acc_v, [d + NPAD], vy)

        pltpu.sync_copy(acc_v, out_hbm.at[wid])

    return run(dst, mm, fx, fy, zeros)


def _reduce_body(p_ref, o_ref):
    o_ref[...] = jnp.sum(p_ref[...], axis=0, keepdims=True)


def _reduce_partials(partials):
    grid = (NP2 // BTC,)
    return pl.pallas_call(
        _reduce_body,
        grid=grid,
        in_specs=[pl.BlockSpec((NW, BTC), lambda i: (0, i))],
        out_specs=pl.BlockSpec((1, BTC), lambda i: (0, i)),
        out_shape=jax.ShapeDtypeStruct((1, NP2), jnp.float32),
    )(partials)


def kernel(node_attr, edge_index, edge_attr, W1, b1, W2, b2, W3, b3):
    node_flat = node_attr.reshape(N)
    src = edge_index[0]
    dst = edge_index[1]
    xs3 = edge_attr[:, 0].reshape(NB, 8, GC)
    ys3 = edge_attr[:, 1].reshape(NB, 8, GC)
    w1t = W1.T
    eye8 = jnp.eye(8, dtype=jnp.float32)
    w1big = jnp.concatenate(
        [jnp.kron(eye8, w1t[:, 0:1]), jnp.kron(eye8, w1t[:, 1:2])],
        axis=1).astype(jnp.bfloat16)
    b1big = jnp.tile(b1, 8).reshape(1024, 1)
    w2tb = W2.T.astype(jnp.bfloat16)
    b2c = b2.reshape(128, 1)
    w3t = W3.T
    b3c = b3.reshape(2, 1)
    zeros = jnp.zeros((NP2,), jnp.float32)

    mm = _mass_product(node_flat, src, dst)
    fx, fy = _edge_mlp(xs3, ys3, w1big, b1big, w2tb, b2c, w3t, b3c)
    partials = _scatter_partials(dst, mm, fx.reshape(E), fy.reshape(E), zeros)
    red = _reduce_partials(partials)
    out = jnp.stack([red[0, :N], red[0, NPAD:NPAD + N]], axis=1)
    return out
